# Initial kernel scaffold; baseline (speedup 1.0000x reference)
import jax
import jax.numpy as jnp
from jax.experimental import pallas as pl


def kernel(x, edge_index, W1, b1, W2, b2, W3, b3, W4, b4, W5, b5, W6, b6):
    raise NotImplementedError("write your pallas kernel here")


# speedup vs baseline: 156.1595x; 156.1595x over previous
import functools

import jax
import jax.numpy as jnp
from jax import lax
from jax.experimental import pallas as pl
from jax.experimental.pallas import tpu as pltpu
from jax.experimental.pallas import tpu_sc as plsc

N = 100000
E = 1600000
L = 16
B = 512
SUB = B // 128
E_PAD = 1638400
TRASH = 8192
N_ACC = N + TRASH
NPT = 6256
NPT_LAST = N - 15 * NPT
BLOCKS_CS = E_PAD // 16 // B
BLOCKS_ES = E_PAD // 32 // B


def _mesh():
    return plsc.VectorSubcoreMesh(core_axis_name="c", subcore_axis_name="s")


_SC_PARAMS = pltpu.CompilerParams(use_tc_tiling_on_sc=False)


def _i32(v):
    return jnp.int32(v)


def _f32(shape):
    return jax.ShapeDtypeStruct(shape, jnp.float32)


def _scatter_block(acc, colv, rows_v):
    for j in range(SUB):
        pltpu.sync_copy(rows_v.at[pl.ds(j * 128, 128)],
                        acc.at[colv.at[_i32(j)]], add=True)


def _gather_block(g_hbm, rowv, rows_v, sem):
    cps = [pltpu.async_copy(g_hbm.at[rowv.at[_i32(j)]],
                            rows_v.at[pl.ds(j * 128, 128)], sem)
           for j in range(SUB)]
    for cp in cps:
        cp.wait()


def _node_copy(src, dst, s, nbase):
    @pl.when(s < 15)
    def _():
        pltpu.sync_copy(src.at[pl.ds(nbase, NPT)],
                        dst.at[pl.ds(nbase, NPT)])

    @pl.when(s == 15)
    def _():
        pltpu.sync_copy(src.at[pl.ds(nbase, NPT_LAST)],
                        dst.at[pl.ds(nbase, NPT_LAST)])


@functools.partial(
    pl.kernel, mesh=_mesh(), compiler_params=_SC_PARAMS,
    out_type=[_f32((N, L)), _f32((N, L))],
    scratch_types=[
        pltpu.VMEM((SUB, 128), jnp.int32),
        pltpu.VMEM((B, L), jnp.float32),
        pltpu.VMEM_SHARED((N_ACC, L), jnp.float32),
        pltpu.SemaphoreType.DMA,
    ],
)
def _sc_degree(ones_h, col_h, p0, p1, colv, rows_v, acc, sem):
    c = lax.axis_index("c")
    s = lax.axis_index("s")
    wid = c * _i32(16) + s
    ebase = wid * _i32(E_PAD // 32 // 128)
    nbase = s * _i32(NPT)
    _node_copy(ones_h, acc, s, nbase)
    pltpu.sync_copy(ones_h.at[pl.ds(0, B)], rows_v)
    plsc.subcore_barrier()

    def body(i, carry):
        off = ebase + i * _i32(SUB)
        pltpu.sync_copy(col_h.at[pl.ds(off, SUB)], colv)
        _scatter_block(acc, colv, rows_v)
        return carry

    lax.fori_loop(_i32(0), _i32(BLOCKS_ES), body, _i32(0))
    plsc.subcore_barrier()

    @pl.when(c == 0)
    def _():
        _node_copy(acc, p0, s, nbase)

    @pl.when(c == 1)
    def _():
        _node_copy(acc, p1, s, nbase)


def _make_spmm_chunksplit(C):
    K = C // 2
    scratch = [
        pltpu.VMEM((SUB, 128), jnp.int32),
        pltpu.VMEM((SUB, 128), jnp.int32),
        pltpu.VMEM((B, L), jnp.float32),
        pltpu.VMEM_SHARED((N_ACC, L), jnp.float32),
        pltpu.SemaphoreType.DMA,
    ]

    @functools.partial(pl.kernel, mesh=_mesh(), compiler_params=_SC_PARAMS,
                       out_type=[_f32((N, L)) for _ in range(C)],
                       scratch_types=scratch)
    def k(*refs):
        gs = refs[:C]
        row_h, col_h = refs[C], refs[C + 1]
        outs = refs[C + 2:2 * C + 2]
        rowv, colv, rows_v, acc, sem = refs[2 * C + 2:]
        c = lax.axis_index("c")
        s = lax.axis_index("s")
        ebase = s * _i32(E_PAD // 16 // 128)
        nbase = s * _i32(NPT)

        for kc in range(K):
            for core in range(2):
                chunk = core * K + kc

                @pl.when(c == core)
                def _(chunk=chunk):
                    _node_copy(gs[chunk], acc, s, nbase)

            plsc.subcore_barrier()

            for core in range(2):
                chunk = core * K + kc

                @pl.when(c == core)
                def _(chunk=chunk):
                    def body(i, carry):
                        off = ebase + i * _i32(SUB)
                        pltpu.sync_copy(row_h.at[pl.ds(off, SUB)], rowv)
                        pltpu.sync_copy(col_h.at[pl.ds(off, SUB)], colv)
                        _gather_block(gs[chunk], rowv, rows_v, sem)
                        _scatter_block(acc, colv, rows_v)
                        return carry

                    lax.fori_loop(_i32(0), _i32(BLOCKS_CS), body, _i32(0))

            plsc.subcore_barrier()

            for core in range(2):
                chunk = core * K + kc

                @pl.when(c == core)
                def _(chunk=chunk):
                    _node_copy(acc, outs[chunk], s, nbase)

            plsc.subcore_barrier()

    return k


@functools.partial(
    pl.kernel, mesh=_mesh(), compiler_params=_SC_PARAMS,
    out_type=[_f32((N, L)), _f32((N, L))],
    scratch_types=[
        pltpu.VMEM((SUB, 128), jnp.int32),
        pltpu.VMEM((SUB, 128), jnp.int32),
        pltpu.VMEM((B, L), jnp.float32),
        pltpu.VMEM_SHARED((N_ACC, L), jnp.float32),
        pltpu.SemaphoreType.DMA,
    ],
)
def _sc_spmm_edgesplit(g, row_h, col_h, p0, p1, rowv, colv, rows_v, acc, sem):
    c = lax.axis_index("c")
    s = lax.axis_index("s")
    wid = c * _i32(16) + s
    ebase = wid * _i32(E_PAD // 32 // 128)
    nbase = s * _i32(NPT)
    _node_copy(g, acc, s, nbase)
    plsc.subcore_barrier()

    def body(i, carry):
        off = ebase + i * _i32(SUB)
        pltpu.sync_copy(row_h.at[pl.ds(off, SUB)], rowv)
        pltpu.sync_copy(col_h.at[pl.ds(off, SUB)], colv)
        _gather_block(g, rowv, rows_v, sem)
        _scatter_block(acc, colv, rows_v)
        return carry

    lax.fori_loop(_i32(0), _i32(BLOCKS_ES), body, _i32(0))
    plsc.subcore_barrier()

    @pl.when(c == 0)
    def _():
        _node_copy(acc, p0, s, nbase)

    @pl.when(c == 1)
    def _():
        _node_copy(acc, p1, s, nbase)


R = 2000
GRID = N // R


def _row_spec(d):
    return pl.BlockSpec((R, d), lambda j: (j, _i32(0)))


def _full_spec(shape):
    return pl.BlockSpec(shape, lambda j: tuple(_i32(0) for _ in shape))


def _tc_first(x, p0, p1, W1):
    dout = W1.shape[1]
    C = dout // L

    def body(x_ref, p0_ref, p1_ref, w_ref, dis_ref, *g_refs):
        deg = p0_ref[...][:, :1] + p1_ref[...][:, :1] - 1.0
        dis = lax.rsqrt(deg)
        dis_ref[...] = dis
        g = dis * jnp.dot(x_ref[...], w_ref[...],
                          precision=lax.Precision.HIGHEST,
                          preferred_element_type=jnp.float32)
        for c in range(C):
            g_refs[c][...] = g[:, c * L:(c + 1) * L]

    return pl.pallas_call(
        body,
        grid=(GRID,),
        in_specs=[_row_spec(x.shape[1]), _row_spec(L), _row_spec(L),
                  _full_spec(W1.shape)],
        out_specs=[_row_spec(1)] + [_row_spec(L)] * C,
        out_shape=[_f32((N, 1))] + [_f32((N, L))] * C,
    )(x, p0, p1, W1)


def _tc_mid(accs, combine, dis, b_prev, W, emit_act=False):
    C_in = len(accs)
    dout = W.shape[1]
    C_out = dout // L

    def body(*refs):
        acc_refs = refs[:C_in]
        dis_ref, b_ref, w_ref = refs[C_in:C_in + 3]
        out_refs = refs[C_in + 3:]
        if combine == "concat":
            prev = jnp.concatenate([a[...] for a in acc_refs], axis=1)
        else:
            prev = acc_refs[0][...] + acc_refs[1][...]
        dis = dis_ref[...]
        act = jnp.maximum(dis * prev + b_ref[...], 0.0)
        off = 0
        if emit_act:
            out_refs[0][...] = act
            off = 1
        g = dis * jnp.dot(act, w_ref[...],
                          precision=lax.Precision.HIGHEST,
                          preferred_element_type=jnp.float32)
        for c in range(C_out):
            out_refs[off + c][...] = g[:, c * L:(c + 1) * L]

    din = b_prev.shape[0]
    out_specs = [_row_spec(L)] * C_out
    out_shape = [_f32((N, L))] * C_out
    if emit_act:
        out_specs = [_row_spec(din)] + out_specs
        out_shape = [_f32((N, din))] + out_shape
    return pl.pallas_call(
        body,
        grid=(GRID,),
        in_specs=[_row_spec(L)] * C_in + [_row_spec(1),
                                          _full_spec((1, din)),
                                          _full_spec(W.shape)],
        out_specs=out_specs,
        out_shape=out_shape,
    )(*accs, dis, b_prev.reshape(1, din), W)


def _tc_final(p0, p1, g, dis, b):
    def body(p0_ref, p1_ref, g_ref, dis_ref, b_ref, out_ref):
        acc = p0_ref[...] + p1_ref[...] - g_ref[...]
        out_ref[...] = jnp.maximum(dis_ref[...] * acc + b_ref[...], 0.0)

    return pl.pallas_call(
        body,
        grid=(GRID,),
        in_specs=[_row_spec(L), _row_spec(L), _row_spec(L), _row_spec(1),
                  _full_spec((1, L))],
        out_specs=_row_spec(L),
        out_shape=_f32((N, L)),
    )(p0, p1, g, dis, b.reshape(1, L))


_spmm4 = _make_spmm_chunksplit(4)
_spmm2 = _make_spmm_chunksplit(2)


def kernel(x, edge_index, W1, b1, W2, b2, W3, b3, W4, b4, W5, b5, W6, b6):
    x = x.astype(jnp.float32)
    W1, b1, W2, b2, W3, b3, W4, b4, W5, b5, W6, b6 = (
        a.astype(jnp.float32)
        for a in (W1, b1, W2, b2, W3, b3, W4, b4, W5, b5, W6, b6))
    pad = E_PAD - E
    row = edge_index[0].astype(jnp.int32)
    col = edge_index[1].astype(jnp.int32)
    fill = jnp.arange(pad, dtype=jnp.int32)
    row_p = jnp.concatenate([row, fill])
    col_p = jnp.concatenate([col, N + (fill % TRASH)])
    col_s, row_s = lax.sort((col_p, row_p), num_keys=1)
    row2d = row_s.reshape(128, E_PAD // 128).T
    col2d = col_s.reshape(128, E_PAD // 128).T
    ones16 = jnp.ones((N, L), jnp.float32)

    d0, d1 = _sc_degree(ones16, col2d)
    dis, *g1 = _tc_first(x, d0, d1, W1)
    a1 = _spmm4(*g1, row2d, col2d)
    g2 = _tc_mid(a1, "concat", dis, b1, W2)
    a2 = _spmm4(*g2, row2d, col2d)
    g3 = _tc_mid(a2, "concat", dis, b2, W3)
    a3 = _spmm2(*g3, row2d, col2d)
    latent, *g4 = _tc_mid(a3, "concat", dis, b3, W4, emit_act=True)
    a4 = _spmm4(*g4, row2d, col2d)
    g5 = _tc_mid(a4, "concat", dis, b4, W5)
    a5 = _spmm4(*g5, row2d, col2d)
    W6p = jnp.pad(W6.astype(jnp.float32), ((0, 0), (0, L - W6.shape[1])))
    g6 = _tc_mid(a5, "concat", dis, b5, W6p)
    p0, p1 = _sc_spmm_edgesplit(g6[0], row2d, col2d)
    b6p = jnp.pad(b6.astype(jnp.float32), (0, L - b6.shape[0]))
    rec = _tc_final(p0, p1, g6[0], dis, b6p)
    return (rec[:, :W6.shape[1]], latent)


# speedup vs baseline: 191.1055x; 1.2238x over previous
import functools

import jax
import jax.numpy as jnp
from jax import lax
from jax.experimental import pallas as pl
from jax.experimental.pallas import tpu as pltpu
from jax.experimental.pallas import tpu_sc as plsc

N = 100000
E = 1600000
L = 16
B = 512
SUB = B // 128
E_PAD = 1638400
TRASH = 8192
N_ACC = N + TRASH
NPT = 6256
NPT_LAST = N - 15 * NPT
BLOCKS_CS = E_PAD // 16 // B
BLOCKS_ES = E_PAD // 32 // B


def _mesh():
    return plsc.VectorSubcoreMesh(core_axis_name="c", subcore_axis_name="s")


_SC_PARAMS = pltpu.CompilerParams(use_tc_tiling_on_sc=False)


def _i32(v):
    return jnp.int32(v)


def _f32(shape):
    return jax.ShapeDtypeStruct(shape, jnp.float32)


def _scatter_block(acc, colv, rows_v):
    for j in range(SUB):
        pltpu.sync_copy(rows_v.at[pl.ds(j * 128, 128)],
                        acc.at[colv.at[_i32(j)]], add=True)


def _gather_block(g_hbm, rowv, rows_v, sem):
    cps = [pltpu.async_copy(g_hbm.at[rowv.at[_i32(j)]],
                            rows_v.at[pl.ds(j * 128, 128)], sem)
           for j in range(SUB)]
    for cp in cps:
        cp.wait()


def _node_copy(src, dst, s, nbase):
    @pl.when(s < 15)
    def _():
        pltpu.sync_copy(src.at[pl.ds(nbase, NPT)],
                        dst.at[pl.ds(nbase, NPT)])

    @pl.when(s == 15)
    def _():
        pltpu.sync_copy(src.at[pl.ds(nbase, NPT_LAST)],
                        dst.at[pl.ds(nbase, NPT_LAST)])


@functools.partial(
    pl.kernel, mesh=_mesh(), compiler_params=_SC_PARAMS,
    out_type=[_f32((N, L)), _f32((N, L))],
    scratch_types=[
        pltpu.VMEM((SUB, 128), jnp.int32),
        pltpu.VMEM((B, L), jnp.float32),
        pltpu.VMEM_SHARED((N_ACC, L), jnp.float32),
        pltpu.SemaphoreType.DMA,
    ],
)
def _sc_degree(ones_h, col_h, p0, p1, colv, rows_v, acc, sem):
    c = lax.axis_index("c")
    s = lax.axis_index("s")
    wid = c * _i32(16) + s
    ebase = wid * _i32(E_PAD // 32 // 128)
    nbase = s * _i32(NPT)
    _node_copy(ones_h, acc, s, nbase)
    pltpu.sync_copy(ones_h.at[pl.ds(0, B)], rows_v)
    plsc.subcore_barrier()

    def body(i, carry):
        off = ebase + i * _i32(SUB)
        pltpu.sync_copy(col_h.at[pl.ds(off, SUB)], colv)
        _scatter_block(acc, colv, rows_v)
        return carry

    lax.fori_loop(_i32(0), _i32(BLOCKS_ES), body, _i32(0))
    plsc.subcore_barrier()

    @pl.when(c == 0)
    def _():
        _node_copy(acc, p0, s, nbase)

    @pl.when(c == 1)
    def _():
        _node_copy(acc, p1, s, nbase)


def _make_spmm_chunksplit(C):
    K = C // 2
    scratch = [
        pltpu.VMEM((SUB, 128), jnp.int32),
        pltpu.VMEM((SUB, 128), jnp.int32),
        pltpu.VMEM((B, L), jnp.float32),
        pltpu.VMEM_SHARED((N_ACC, L), jnp.float32),
        pltpu.SemaphoreType.DMA,
    ]

    @functools.partial(pl.kernel, mesh=_mesh(), compiler_params=_SC_PARAMS,
                       out_type=[_f32((N, L)) for _ in range(C)],
                       scratch_types=scratch)
    def k(*refs):
        gs = refs[:C]
        row_h, col_h = refs[C], refs[C + 1]
        outs = refs[C + 2:2 * C + 2]
        rowv, colv, rows_v, acc, sem = refs[2 * C + 2:]
        c = lax.axis_index("c")
        s = lax.axis_index("s")
        ebase = s * _i32(E_PAD // 16 // 128)
        nbase = s * _i32(NPT)

        for kc in range(K):
            for core in range(2):
                chunk = core * K + kc

                @pl.when(c == core)
                def _(chunk=chunk):
                    _node_copy(gs[chunk], acc, s, nbase)

            plsc.subcore_barrier()

            for core in range(2):
                chunk = core * K + kc

                @pl.when(c == core)
                def _(chunk=chunk):
                    def body(i, carry):
                        off = ebase + i * _i32(SUB)
                        pltpu.sync_copy(row_h.at[pl.ds(off, SUB)], rowv)
                        pltpu.sync_copy(col_h.at[pl.ds(off, SUB)], colv)
                        _gather_block(gs[chunk], rowv, rows_v, sem)
                        _scatter_block(acc, colv, rows_v)
                        return carry

                    lax.fori_loop(_i32(0), _i32(BLOCKS_CS), body, _i32(0))

            plsc.subcore_barrier()

            for core in range(2):
                chunk = core * K + kc

                @pl.when(c == core)
                def _(chunk=chunk):
                    _node_copy(acc, outs[chunk], s, nbase)

            plsc.subcore_barrier()

    return k


@functools.partial(
    pl.kernel, mesh=_mesh(), compiler_params=_SC_PARAMS,
    out_type=[_f32((N, L)), _f32((N, L))],
    scratch_types=[
        pltpu.VMEM((SUB, 128), jnp.int32),
        pltpu.VMEM((SUB, 128), jnp.int32),
        pltpu.VMEM((B, L), jnp.float32),
        pltpu.VMEM_SHARED((N_ACC, L), jnp.float32),
        pltpu.SemaphoreType.DMA,
    ],
)
def _sc_spmm_edgesplit(g, row_h, col_h, p0, p1, rowv, colv, rows_v, acc, sem):
    c = lax.axis_index("c")
    s = lax.axis_index("s")
    wid = c * _i32(16) + s
    ebase = wid * _i32(E_PAD // 32 // 128)
    nbase = s * _i32(NPT)
    _node_copy(g, acc, s, nbase)
    plsc.subcore_barrier()

    def body(i, carry):
        off = ebase + i * _i32(SUB)
        pltpu.sync_copy(row_h.at[pl.ds(off, SUB)], rowv)
        pltpu.sync_copy(col_h.at[pl.ds(off, SUB)], colv)
        _gather_block(g, rowv, rows_v, sem)
        _scatter_block(acc, colv, rows_v)
        return carry

    lax.fori_loop(_i32(0), _i32(BLOCKS_ES), body, _i32(0))
    plsc.subcore_barrier()

    @pl.when(c == 0)
    def _():
        _node_copy(acc, p0, s, nbase)

    @pl.when(c == 1)
    def _():
        _node_copy(acc, p1, s, nbase)


R = 2000
GRID = N // R


def _row_spec(d):
    return pl.BlockSpec((R, d), lambda j: (j, _i32(0)))


def _full_spec(shape):
    return pl.BlockSpec(shape, lambda j: tuple(_i32(0) for _ in shape))


def _tc_first(x, p0, p1, W1):
    dout = W1.shape[1]
    C = dout // L

    def body(x_ref, p0_ref, p1_ref, w_ref, dis_ref, *g_refs):
        deg = p0_ref[...][:, :1] + p1_ref[...][:, :1] - 1.0
        dis = lax.rsqrt(deg)
        dis_ref[...] = dis
        g = dis * jnp.dot(x_ref[...], w_ref[...],
                          precision=lax.Precision.HIGHEST,
                          preferred_element_type=jnp.float32)
        for c in range(C):
            g_refs[c][...] = g[:, c * L:(c + 1) * L]

    return pl.pallas_call(
        body,
        grid=(GRID,),
        in_specs=[_row_spec(x.shape[1]), _row_spec(L), _row_spec(L),
                  _full_spec(W1.shape)],
        out_specs=[_row_spec(1)] + [_row_spec(L)] * C,
        out_shape=[_f32((N, 1))] + [_f32((N, L))] * C,
    )(x, p0, p1, W1)


def _tc_mid(accs, combine, dis, b_prev, W, emit_act=False):
    C_in = len(accs)
    dout = W.shape[1]
    C_out = dout // L

    def body(*refs):
        acc_refs = refs[:C_in]
        dis_ref, b_ref, w_ref = refs[C_in:C_in + 3]
        out_refs = refs[C_in + 3:]
        if combine == "concat":
            prev = jnp.concatenate([a[...] for a in acc_refs], axis=1)
        else:
            prev = acc_refs[0][...] + acc_refs[1][...]
        dis = dis_ref[...]
        act = jnp.maximum(dis * prev + b_ref[...], 0.0)
        off = 0
        if emit_act:
            out_refs[0][...] = act
            off = 1
        g = dis * jnp.dot(act, w_ref[...],
                          precision=lax.Precision.HIGHEST,
                          preferred_element_type=jnp.float32)
        for c in range(C_out):
            out_refs[off + c][...] = g[:, c * L:(c + 1) * L]

    din = b_prev.shape[0]
    out_specs = [_row_spec(L)] * C_out
    out_shape = [_f32((N, L))] * C_out
    if emit_act:
        out_specs = [_row_spec(din)] + out_specs
        out_shape = [_f32((N, din))] + out_shape
    return pl.pallas_call(
        body,
        grid=(GRID,),
        in_specs=[_row_spec(L)] * C_in + [_row_spec(1),
                                          _full_spec((1, din)),
                                          _full_spec(W.shape)],
        out_specs=out_specs,
        out_shape=out_shape,
    )(*accs, dis, b_prev.reshape(1, din), W)


def _tc_final(p0, p1, g, dis, b):
    def body(p0_ref, p1_ref, g_ref, dis_ref, b_ref, out_ref):
        acc = p0_ref[...] + p1_ref[...] - g_ref[...]
        out_ref[...] = jnp.maximum(dis_ref[...] * acc + b_ref[...], 0.0)

    return pl.pallas_call(
        body,
        grid=(GRID,),
        in_specs=[_row_spec(L), _row_spec(L), _row_spec(L), _row_spec(1),
                  _full_spec((1, L))],
        out_specs=_row_spec(L),
        out_shape=_f32((N, L)),
    )(p0, p1, g, dis, b.reshape(1, L))


_spmm4 = _make_spmm_chunksplit(4)
_spmm2 = _make_spmm_chunksplit(2)


def kernel(x, edge_index, W1, b1, W2, b2, W3, b3, W4, b4, W5, b5, W6, b6):
    x = x.astype(jnp.float32)
    W1, b1, W2, b2, W3, b3, W4, b4, W5, b5, W6, b6 = (
        a.astype(jnp.float32)
        for a in (W1, b1, W2, b2, W3, b3, W4, b4, W5, b5, W6, b6))
    pad = E_PAD - E
    row = edge_index[0].astype(jnp.int32)
    col = edge_index[1].astype(jnp.int32)
    fill = jnp.arange(pad, dtype=jnp.int32)
    row2d = jnp.concatenate([row, fill]).reshape(E_PAD // 128, 128)
    col2d = jnp.concatenate([col, N + (fill % TRASH)]).reshape(
        E_PAD // 128, 128)
    ones16 = jnp.ones((N, L), jnp.float32)

    d0, d1 = _sc_degree(ones16, col2d)
    dis, *g1 = _tc_first(x, d0, d1, W1)
    a1 = _spmm4(*g1, row2d, col2d)
    g2 = _tc_mid(a1, "concat", dis, b1, W2)
    a2 = _spmm4(*g2, row2d, col2d)
    g3 = _tc_mid(a2, "concat", dis, b2, W3)
    a3 = _spmm2(*g3, row2d, col2d)
    latent, *g4 = _tc_mid(a3, "concat", dis, b3, W4, emit_act=True)
    a4 = _spmm4(*g4, row2d, col2d)
    g5 = _tc_mid(a4, "concat", dis, b4, W5)
    a5 = _spmm4(*g5, row2d, col2d)
    W6p = jnp.pad(W6.astype(jnp.float32), ((0, 0), (0, L - W6.shape[1])))
    g6 = _tc_mid(a5, "concat", dis, b5, W6p)
    p0, p1 = _sc_spmm_edgesplit(g6[0], row2d, col2d)
    b6p = jnp.pad(b6.astype(jnp.float32), (0, L - b6.shape[0]))
    rec = _tc_final(p0, p1, g6[0], dis, b6p)
    return (rec[:, :W6.shape[1]], latent)


# speedup vs baseline: 249.0286x; 1.3031x over previous
import functools

import jax
import jax.numpy as jnp
from jax import lax
from jax.experimental import pallas as pl
from jax.experimental.pallas import tpu as pltpu
from jax.experimental.pallas import tpu_sc as plsc

N = 100000
E = 1600000
L = 16
B = 512
SUB = B // 128
E_PAD = 1638400
TRASH = 8192
N_ACC = N + TRASH
NPT = 6256
NPT_LAST = N - 15 * NPT
BLOCKS_CS = E_PAD // 16 // B
BLOCKS_ES = E_PAD // 32 // B


def _mesh():
    return plsc.VectorSubcoreMesh(core_axis_name="c", subcore_axis_name="s")


_SC_PARAMS = pltpu.CompilerParams(use_tc_tiling_on_sc=False)


def _i32(v):
    return jnp.int32(v)


def _f32(shape):
    return jax.ShapeDtypeStruct(shape, jnp.float32)


def _scatter_block(acc, colv, rows_v):
    for j in range(SUB):
        pltpu.sync_copy(rows_v.at[pl.ds(j * 128, 128)],
                        acc.at[colv.at[_i32(j)]], add=True)


def _drain_scatter(acc, colv, rows_v, sem):
    for j in range(SUB):
        pltpu.make_async_copy(rows_v.at[pl.ds(j * 128, 128)],
                              acc.at[colv.at[_i32(j)]], sem).wait()


def _edge_pass(g_hbm, row_h, col_h, acc, ebase, nblocks, bufs):
    rowv, colv, rows, semi, semg, sems = bufs

    def super_body(g, carry):
        for p in range(2):
            b = g * _i32(2) + _i32(p)
            off = ebase + b * _i32(SUB)

            @pl.when(g > 0)
            def _(p=p):
                _drain_scatter(acc, colv[p], rows[p], sems[p])

            cpr = pltpu.async_copy(row_h.at[pl.ds(off, SUB)], rowv[p], semi)
            cpc = pltpu.async_copy(col_h.at[pl.ds(off, SUB)], colv[p], semi)
            cpr.wait()
            cpc.wait()
            cps = [pltpu.async_copy(g_hbm.at[rowv[p].at[_i32(j)]],
                                    rows[p].at[pl.ds(j * 128, 128)], semg)
                   for j in range(SUB)]
            for cp in cps:
                cp.wait()
            for j in range(SUB):
                pltpu.async_copy(rows[p].at[pl.ds(j * 128, 128)],
                                 acc.at[colv[p].at[_i32(j)]], sems[p],
                                 add=True)
        return carry

    lax.fori_loop(_i32(0), _i32(nblocks // 2), super_body, _i32(0))
    for p in range(2):
        _drain_scatter(acc, colv[p], rows[p], sems[p])


_PIPE_SCRATCH = [
    pltpu.VMEM((SUB, 128), jnp.int32),
    pltpu.VMEM((SUB, 128), jnp.int32),
    pltpu.VMEM((SUB, 128), jnp.int32),
    pltpu.VMEM((SUB, 128), jnp.int32),
    pltpu.VMEM((B, L), jnp.float32),
    pltpu.VMEM((B, L), jnp.float32),
    pltpu.VMEM_SHARED((N_ACC, L), jnp.float32),
    pltpu.SemaphoreType.DMA,
    pltpu.SemaphoreType.DMA,
    pltpu.SemaphoreType.DMA,
    pltpu.SemaphoreType.DMA,
]


def _pipe_bufs(refs):
    r0, r1, c0, c1, w0, w1, acc, semi, semg, s0, s1 = refs
    return acc, ((r0, r1), (c0, c1), (w0, w1), semi, semg, (s0, s1))


def _node_copy(src, dst, s, nbase):
    @pl.when(s < 15)
    def _():
        pltpu.sync_copy(src.at[pl.ds(nbase, NPT)],
                        dst.at[pl.ds(nbase, NPT)])

    @pl.when(s == 15)
    def _():
        pltpu.sync_copy(src.at[pl.ds(nbase, NPT_LAST)],
                        dst.at[pl.ds(nbase, NPT_LAST)])


@functools.partial(
    pl.kernel, mesh=_mesh(), compiler_params=_SC_PARAMS,
    out_type=[_f32((N, L)), _f32((N, L))],
    scratch_types=[
        pltpu.VMEM((SUB, 128), jnp.int32),
        pltpu.VMEM((B, L), jnp.float32),
        pltpu.VMEM_SHARED((N_ACC, L), jnp.float32),
        pltpu.SemaphoreType.DMA,
    ],
)
def _sc_degree(ones_h, col_h, p0, p1, colv, rows_v, acc, sem):
    c = lax.axis_index("c")
    s = lax.axis_index("s")
    wid = c * _i32(16) + s
    ebase = wid * _i32(E_PAD // 32 // 128)
    nbase = s * _i32(NPT)
    _node_copy(ones_h, acc, s, nbase)
    pltpu.sync_copy(ones_h.at[pl.ds(0, B)], rows_v)
    plsc.subcore_barrier()

    def body(i, carry):
        off = ebase + i * _i32(SUB)
        pltpu.sync_copy(col_h.at[pl.ds(off, SUB)], colv)
        _scatter_block(acc, colv, rows_v)
        return carry

    lax.fori_loop(_i32(0), _i32(BLOCKS_ES), body, _i32(0))
    plsc.subcore_barrier()

    @pl.when(c == 0)
    def _():
        _node_copy(acc, p0, s, nbase)

    @pl.when(c == 1)
    def _():
        _node_copy(acc, p1, s, nbase)


def _make_spmm_chunksplit(C):
    K = C // 2

    @functools.partial(pl.kernel, mesh=_mesh(), compiler_params=_SC_PARAMS,
                       out_type=[_f32((N, L)) for _ in range(C)],
                       scratch_types=list(_PIPE_SCRATCH))
    def k(*refs):
        gs = refs[:C]
        row_h, col_h = refs[C], refs[C + 1]
        outs = refs[C + 2:2 * C + 2]
        acc, bufs = _pipe_bufs(refs[2 * C + 2:])
        c = lax.axis_index("c")
        s = lax.axis_index("s")
        ebase = s * _i32(E_PAD // 16 // 128)
        nbase = s * _i32(NPT)

        for kc in range(K):
            for core in range(2):
                chunk = core * K + kc

                @pl.when(c == core)
                def _(chunk=chunk):
                    _node_copy(gs[chunk], acc, s, nbase)

            plsc.subcore_barrier()

            for core in range(2):
                chunk = core * K + kc

                @pl.when(c == core)
                def _(chunk=chunk):
                    _edge_pass(gs[chunk], row_h, col_h, acc, ebase,
                               BLOCKS_CS, bufs)

            plsc.subcore_barrier()

            for core in range(2):
                chunk = core * K + kc

                @pl.when(c == core)
                def _(chunk=chunk):
                    _node_copy(acc, outs[chunk], s, nbase)

            plsc.subcore_barrier()

    return k


@functools.partial(
    pl.kernel, mesh=_mesh(), compiler_params=_SC_PARAMS,
    out_type=[_f32((N, L)), _f32((N, L))],
    scratch_types=list(_PIPE_SCRATCH),
)
def _sc_spmm_edgesplit(g, row_h, col_h, p0, p1, *scr):
    acc, bufs = _pipe_bufs(scr)
    c = lax.axis_index("c")
    s = lax.axis_index("s")
    wid = c * _i32(16) + s
    ebase = wid * _i32(E_PAD // 32 // 128)
    nbase = s * _i32(NPT)
    _node_copy(g, acc, s, nbase)
    plsc.subcore_barrier()
    _edge_pass(g, row_h, col_h, acc, ebase, BLOCKS_ES, bufs)
    plsc.subcore_barrier()

    @pl.when(c == 0)
    def _():
        _node_copy(acc, p0, s, nbase)

    @pl.when(c == 1)
    def _():
        _node_copy(acc, p1, s, nbase)


R = 2000
GRID = N // R


def _row_spec(d):
    return pl.BlockSpec((R, d), lambda j: (j, _i32(0)))


def _full_spec(shape):
    return pl.BlockSpec(shape, lambda j: tuple(_i32(0) for _ in shape))


def _tc_first(x, p0, p1, W1):
    dout = W1.shape[1]
    C = dout // L

    def body(x_ref, p0_ref, p1_ref, w_ref, dis_ref, *g_refs):
        deg = p0_ref[...][:, :1] + p1_ref[...][:, :1] - 1.0
        dis = lax.rsqrt(deg)
        dis_ref[...] = dis
        g = dis * jnp.dot(x_ref[...], w_ref[...],
                          precision=lax.Precision.HIGHEST,
                          preferred_element_type=jnp.float32)
        for c in range(C):
            g_refs[c][...] = g[:, c * L:(c + 1) * L]

    return pl.pallas_call(
        body,
        grid=(GRID,),
        in_specs=[_row_spec(x.shape[1]), _row_spec(L), _row_spec(L),
                  _full_spec(W1.shape)],
        out_specs=[_row_spec(1)] + [_row_spec(L)] * C,
        out_shape=[_f32((N, 1))] + [_f32((N, L))] * C,
    )(x, p0, p1, W1)


def _tc_mid(accs, combine, dis, b_prev, W, emit_act=False):
    C_in = len(accs)
    dout = W.shape[1]
    C_out = dout // L

    def body(*refs):
        acc_refs = refs[:C_in]
        dis_ref, b_ref, w_ref = refs[C_in:C_in + 3]
        out_refs = refs[C_in + 3:]
        if combine == "concat":
            prev = jnp.concatenate([a[...] for a in acc_refs], axis=1)
        else:
            prev = acc_refs[0][...] + acc_refs[1][...]
        dis = dis_ref[...]
        act = jnp.maximum(dis * prev + b_ref[...], 0.0)
        off = 0
        if emit_act:
            out_refs[0][...] = act
            off = 1
        g = dis * jnp.dot(act, w_ref[...],
                          precision=lax.Precision.HIGHEST,
                          preferred_element_type=jnp.float32)
        for c in range(C_out):
            out_refs[off + c][...] = g[:, c * L:(c + 1) * L]

    din = b_prev.shape[0]
    out_specs = [_row_spec(L)] * C_out
    out_shape = [_f32((N, L))] * C_out
    if emit_act:
        out_specs = [_row_spec(din)] + out_specs
        out_shape = [_f32((N, din))] + out_shape
    return pl.pallas_call(
        body,
        grid=(GRID,),
        in_specs=[_row_spec(L)] * C_in + [_row_spec(1),
                                          _full_spec((1, din)),
                                          _full_spec(W.shape)],
        out_specs=out_specs,
        out_shape=out_shape,
    )(*accs, dis, b_prev.reshape(1, din), W)


def _tc_final(p0, p1, g, dis, b):
    def body(p0_ref, p1_ref, g_ref, dis_ref, b_ref, out_ref):
        acc = p0_ref[...] + p1_ref[...] - g_ref[...]
        out_ref[...] = jnp.maximum(dis_ref[...] * acc + b_ref[...], 0.0)

    return pl.pallas_call(
        body,
        grid=(GRID,),
        in_specs=[_row_spec(L), _row_spec(L), _row_spec(L), _row_spec(1),
                  _full_spec((1, L))],
        out_specs=_row_spec(L),
        out_shape=_f32((N, L)),
    )(p0, p1, g, dis, b.reshape(1, L))


_spmm4 = _make_spmm_chunksplit(4)
_spmm2 = _make_spmm_chunksplit(2)


def kernel(x, edge_index, W1, b1, W2, b2, W3, b3, W4, b4, W5, b5, W6, b6):
    x = x.astype(jnp.float32)
    W1, b1, W2, b2, W3, b3, W4, b4, W5, b5, W6, b6 = (
        a.astype(jnp.float32)
        for a in (W1, b1, W2, b2, W3, b3, W4, b4, W5, b5, W6, b6))
    pad = E_PAD - E
    row = edge_index[0].astype(jnp.int32)
    col = edge_index[1].astype(jnp.int32)
    fill = jnp.arange(pad, dtype=jnp.int32)
    row2d = jnp.concatenate([row, fill]).reshape(E_PAD // 128, 128)
    col2d = jnp.concatenate([col, N + (fill % TRASH)]).reshape(
        E_PAD // 128, 128)
    ones16 = jnp.ones((N, L), jnp.float32)

    d0, d1 = _sc_degree(ones16, col2d)
    dis, *g1 = _tc_first(x, d0, d1, W1)
    a1 = _spmm4(*g1, row2d, col2d)
    g2 = _tc_mid(a1, "concat", dis, b1, W2)
    a2 = _spmm4(*g2, row2d, col2d)
    g3 = _tc_mid(a2, "concat", dis, b2, W3)
    a3 = _spmm2(*g3, row2d, col2d)
    latent, *g4 = _tc_mid(a3, "concat", dis, b3, W4, emit_act=True)
    a4 = _spmm4(*g4, row2d, col2d)
    g5 = _tc_mid(a4, "concat", dis, b4, W5)
    a5 = _spmm4(*g5, row2d, col2d)
    W6p = jnp.pad(W6.astype(jnp.float32), ((0, 0), (0, L - W6.shape[1])))
    g6 = _tc_mid(a5, "concat", dis, b5, W6p)
    p0, p1 = _sc_spmm_edgesplit(g6[0], row2d, col2d)
    b6p = jnp.pad(b6.astype(jnp.float32), (0, L - b6.shape[0]))
    rec = _tc_final(p0, p1, g6[0], dis, b6p)
    return (rec[:, :W6.shape[1]], latent)


# speedup vs baseline: 299.3076x; 1.2019x over previous
import functools

import jax
import jax.numpy as jnp
from jax import lax
from jax.experimental import pallas as pl
from jax.experimental.pallas import tpu as pltpu
from jax.experimental.pallas import tpu_sc as plsc

N = 100000
E = 1600000
L = 16
B = 512
SUB = B // 128
E_PAD = 1638400
TRASH = 2048
N_ACC = N + TRASH
GB = 4
NPT = 6256
NPT_LAST = N - 15 * NPT
BLOCKS_CS = E_PAD // 16 // B
BLOCKS_ES = E_PAD // 32 // B


def _mesh():
    return plsc.VectorSubcoreMesh(core_axis_name="c", subcore_axis_name="s")


_SC_PARAMS = pltpu.CompilerParams(use_tc_tiling_on_sc=False)


def _i32(v):
    return jnp.int32(v)


def _f32(shape):
    return jax.ShapeDtypeStruct(shape, jnp.float32)


def _scatter_block(acc, colv, rows_v):
    for j in range(SUB):
        pltpu.sync_copy(rows_v.at[pl.ds(j * 128, 128)],
                        acc.at[colv.at[_i32(j)]], add=True)


def _drain_scatter(acc, colv, rows_v, sem):
    for j in range(SUB):
        pltpu.make_async_copy(rows_v.at[pl.ds(j * 128, 128)],
                              acc.at[colv.at[_i32(j)]], sem).wait()


def _edge_pass(g_hbm, row_h, col_h, acc, ebase, nblocks, bufs):
    rowg, colg, rows, semi, semg, sems = bufs
    ngroups = nblocks // GB

    def load_group(gi, q):
        goff = ebase + gi * _i32(GB * SUB)
        pltpu.async_copy(row_h.at[pl.ds(goff, GB * SUB)], rowg[q], semi)
        pltpu.async_copy(col_h.at[pl.ds(goff, GB * SUB)], colg[q], semi)

    load_group(_i32(0), 0)

    def group_body(gi, carry):
        q = lax.rem(gi, _i32(2))
        for qs in range(2):
            @pl.when(q == qs)
            def _(qs=qs):
                pltpu.make_async_copy(row_h.at[pl.ds(ebase, GB * SUB)],
                                      rowg[qs], semi).wait()
                pltpu.make_async_copy(col_h.at[pl.ds(ebase, GB * SUB)],
                                      colg[qs], semi).wait()

                @pl.when(gi < _i32(ngroups - 1))
                def _():
                    load_group(gi + _i32(1), 1 - qs)

                for k in range(GB):
                    p = k % 2
                    if k < 2:
                        @pl.when(gi > 0)
                        def _(p=p, qs=qs):
                            _drain_scatter(acc, colg[qs], rows[p], sems[p])
                    else:
                        _drain_scatter(acc, colg[qs], rows[p], sems[p])
                    cps = [pltpu.async_copy(
                               g_hbm.at[rowg[qs].at[_i32(k * SUB + j)]],
                               rows[p].at[pl.ds(j * 128, 128)], semg)
                           for j in range(SUB)]
                    for cp in cps:
                        cp.wait()
                    for j in range(SUB):
                        pltpu.async_copy(rows[p].at[pl.ds(j * 128, 128)],
                                         acc.at[colg[qs].at[_i32(k * SUB + j)]],
                                         sems[p], add=True)
        return carry

    lax.fori_loop(_i32(0), _i32(ngroups), group_body, _i32(0))
    for p in range(2):
        _drain_scatter(acc, colg[0], rows[p], sems[p])


_PIPE_SCRATCH = [
    pltpu.VMEM((GB * SUB, 128), jnp.int32),
    pltpu.VMEM((GB * SUB, 128), jnp.int32),
    pltpu.VMEM((GB * SUB, 128), jnp.int32),
    pltpu.VMEM((GB * SUB, 128), jnp.int32),
    pltpu.VMEM((B, L), jnp.float32),
    pltpu.VMEM((B, L), jnp.float32),
    pltpu.VMEM_SHARED((N_ACC, L), jnp.float32),
    pltpu.SemaphoreType.DMA,
    pltpu.SemaphoreType.DMA,
    pltpu.SemaphoreType.DMA,
    pltpu.SemaphoreType.DMA,
]


def _pipe_bufs(refs):
    r0, r1, c0, c1, w0, w1, acc, semi, semg, s0, s1 = refs
    return acc, ((r0, r1), (c0, c1), (w0, w1), semi, semg, (s0, s1))


def _node_copy(src, dst, s, nbase):
    @pl.when(s < 15)
    def _():
        pltpu.sync_copy(src.at[pl.ds(nbase, NPT)],
                        dst.at[pl.ds(nbase, NPT)])

    @pl.when(s == 15)
    def _():
        pltpu.sync_copy(src.at[pl.ds(nbase, NPT_LAST)],
                        dst.at[pl.ds(nbase, NPT_LAST)])


@functools.partial(
    pl.kernel, mesh=_mesh(), compiler_params=_SC_PARAMS,
    out_type=[_f32((N, L)), _f32((N, L))],
    scratch_types=[
        pltpu.VMEM((SUB, 128), jnp.int32),
        pltpu.VMEM((B, L), jnp.float32),
        pltpu.VMEM_SHARED((N_ACC, L), jnp.float32),
        pltpu.SemaphoreType.DMA,
    ],
)
def _sc_degree(ones_h, col_h, p0, p1, colv, rows_v, acc, sem):
    c = lax.axis_index("c")
    s = lax.axis_index("s")
    wid = c * _i32(16) + s
    ebase = wid * _i32(E_PAD // 32 // 128)
    nbase = s * _i32(NPT)
    _node_copy(ones_h, acc, s, nbase)
    pltpu.sync_copy(ones_h.at[pl.ds(0, B)], rows_v)
    plsc.subcore_barrier()

    def body(i, carry):
        off = ebase + i * _i32(SUB)
        pltpu.sync_copy(col_h.at[pl.ds(off, SUB)], colv)
        _scatter_block(acc, colv, rows_v)
        return carry

    lax.fori_loop(_i32(0), _i32(BLOCKS_ES), body, _i32(0))
    plsc.subcore_barrier()

    @pl.when(c == 0)
    def _():
        _node_copy(acc, p0, s, nbase)

    @pl.when(c == 1)
    def _():
        _node_copy(acc, p1, s, nbase)


def _make_spmm_chunksplit(C):
    K = C // 2

    @functools.partial(pl.kernel, mesh=_mesh(), compiler_params=_SC_PARAMS,
                       out_type=[_f32((N, L)) for _ in range(C)],
                       scratch_types=list(_PIPE_SCRATCH))
    def k(*refs):
        gs = refs[:C]
        row_h, col_h = refs[C], refs[C + 1]
        outs = refs[C + 2:2 * C + 2]
        acc, bufs = _pipe_bufs(refs[2 * C + 2:])
        c = lax.axis_index("c")
        s = lax.axis_index("s")
        ebase = s * _i32(E_PAD // 16 // 128)
        nbase = s * _i32(NPT)

        for kc in range(K):
            for core in range(2):
                chunk = core * K + kc

                @pl.when(c == core)
                def _(chunk=chunk):
                    _node_copy(gs[chunk], acc, s, nbase)

            plsc.subcore_barrier()

            for core in range(2):
                chunk = core * K + kc

                @pl.when(c == core)
                def _(chunk=chunk):
                    _edge_pass(gs[chunk], row_h, col_h, acc, ebase,
                               BLOCKS_CS, bufs)

            plsc.subcore_barrier()

            for core in range(2):
                chunk = core * K + kc

                @pl.when(c == core)
                def _(chunk=chunk):
                    _node_copy(acc, outs[chunk], s, nbase)

            plsc.subcore_barrier()

    return k


@functools.partial(
    pl.kernel, mesh=_mesh(), compiler_params=_SC_PARAMS,
    out_type=[_f32((N, L)), _f32((N, L))],
    scratch_types=list(_PIPE_SCRATCH),
)
def _sc_spmm_edgesplit(g, row_h, col_h, p0, p1, *scr):
    acc, bufs = _pipe_bufs(scr)
    c = lax.axis_index("c")
    s = lax.axis_index("s")
    wid = c * _i32(16) + s
    ebase = wid * _i32(E_PAD // 32 // 128)
    nbase = s * _i32(NPT)
    _node_copy(g, acc, s, nbase)
    plsc.subcore_barrier()
    _edge_pass(g, row_h, col_h, acc, ebase, BLOCKS_ES, bufs)
    plsc.subcore_barrier()

    @pl.when(c == 0)
    def _():
        _node_copy(acc, p0, s, nbase)

    @pl.when(c == 1)
    def _():
        _node_copy(acc, p1, s, nbase)


R = 2000
GRID = N // R


def _row_spec(d):
    return pl.BlockSpec((R, d), lambda j: (j, _i32(0)))


def _full_spec(shape):
    return pl.BlockSpec(shape, lambda j: tuple(_i32(0) for _ in shape))


def _tc_first(x, p0, p1, W1):
    dout = W1.shape[1]
    C = dout // L

    def body(x_ref, p0_ref, p1_ref, w_ref, dis_ref, *g_refs):
        deg = p0_ref[...][:, :1] + p1_ref[...][:, :1] - 1.0
        dis = lax.rsqrt(deg)
        dis_ref[...] = dis
        g = dis * jnp.dot(x_ref[...], w_ref[...],
                          precision=lax.Precision.HIGHEST,
                          preferred_element_type=jnp.float32)
        for c in range(C):
            g_refs[c][...] = g[:, c * L:(c + 1) * L]

    return pl.pallas_call(
        body,
        grid=(GRID,),
        in_specs=[_row_spec(x.shape[1]), _row_spec(L), _row_spec(L),
                  _full_spec(W1.shape)],
        out_specs=[_row_spec(1)] + [_row_spec(L)] * C,
        out_shape=[_f32((N, 1))] + [_f32((N, L))] * C,
    )(x, p0, p1, W1)


def _tc_mid(accs, combine, dis, b_prev, W, emit_act=False):
    C_in = len(accs)
    dout = W.shape[1]
    C_out = dout // L

    def body(*refs):
        acc_refs = refs[:C_in]
        dis_ref, b_ref, w_ref = refs[C_in:C_in + 3]
        out_refs = refs[C_in + 3:]
        if combine == "concat":
            prev = jnp.concatenate([a[...] for a in acc_refs], axis=1)
        else:
            prev = acc_refs[0][...] + acc_refs[1][...]
        dis = dis_ref[...]
        act = jnp.maximum(dis * prev + b_ref[...], 0.0)
        off = 0
        if emit_act:
            out_refs[0][...] = act
            off = 1
        g = dis * jnp.dot(act, w_ref[...],
                          precision=lax.Precision.HIGHEST,
                          preferred_element_type=jnp.float32)
        for c in range(C_out):
            out_refs[off + c][...] = g[:, c * L:(c + 1) * L]

    din = b_prev.shape[0]
    out_specs = [_row_spec(L)] * C_out
    out_shape = [_f32((N, L))] * C_out
    if emit_act:
        out_specs = [_row_spec(din)] + out_specs
        out_shape = [_f32((N, din))] + out_shape
    return pl.pallas_call(
        body,
        grid=(GRID,),
        in_specs=[_row_spec(L)] * C_in + [_row_spec(1),
                                          _full_spec((1, din)),
                                          _full_spec(W.shape)],
        out_specs=out_specs,
        out_shape=out_shape,
    )(*accs, dis, b_prev.reshape(1, din), W)


def _tc_final(p0, p1, g, dis, b):
    def body(p0_ref, p1_ref, g_ref, dis_ref, b_ref, out_ref):
        acc = p0_ref[...] + p1_ref[...] - g_ref[...]
        out_ref[...] = jnp.maximum(dis_ref[...] * acc + b_ref[...], 0.0)

    return pl.pallas_call(
        body,
        grid=(GRID,),
        in_specs=[_row_spec(L), _row_spec(L), _row_spec(L), _row_spec(1),
                  _full_spec((1, L))],
        out_specs=_row_spec(L),
        out_shape=_f32((N, L)),
    )(p0, p1, g, dis, b.reshape(1, L))


_spmm4 = _make_spmm_chunksplit(4)
_spmm2 = _make_spmm_chunksplit(2)


def kernel(x, edge_index, W1, b1, W2, b2, W3, b3, W4, b4, W5, b5, W6, b6):
    x = x.astype(jnp.float32)
    W1, b1, W2, b2, W3, b3, W4, b4, W5, b5, W6, b6 = (
        a.astype(jnp.float32)
        for a in (W1, b1, W2, b2, W3, b3, W4, b4, W5, b5, W6, b6))
    pad = E_PAD - E
    row = edge_index[0].astype(jnp.int32)
    col = edge_index[1].astype(jnp.int32)
    fill = jnp.arange(pad, dtype=jnp.int32)
    row2d = jnp.concatenate([row, fill]).reshape(E_PAD // 128, 128)
    col2d = jnp.concatenate([col, N + (fill % TRASH)]).reshape(
        E_PAD // 128, 128)
    ones16 = jnp.ones((N, L), jnp.float32)

    d0, d1 = _sc_degree(ones16, col2d)
    dis, *g1 = _tc_first(x, d0, d1, W1)
    a1 = _spmm4(*g1, row2d, col2d)
    g2 = _tc_mid(a1, "concat", dis, b1, W2)
    a2 = _spmm4(*g2, row2d, col2d)
    g3 = _tc_mid(a2, "concat", dis, b2, W3)
    a3 = _spmm2(*g3, row2d, col2d)
    latent, *g4 = _tc_mid(a3, "concat", dis, b3, W4, emit_act=True)
    a4 = _spmm4(*g4, row2d, col2d)
    g5 = _tc_mid(a4, "concat", dis, b4, W5)
    a5 = _spmm4(*g5, row2d, col2d)
    W6p = jnp.pad(W6.astype(jnp.float32), ((0, 0), (0, L - W6.shape[1])))
    g6 = _tc_mid(a5, "concat", dis, b5, W6p)
    p0, p1 = _sc_spmm_edgesplit(g6[0], row2d, col2d)
    b6p = jnp.pad(b6.astype(jnp.float32), (0, L - b6.shape[0]))
    rec = _tc_final(p0, p1, g6[0], dis, b6p)
    return (rec[:, :W6.shape[1]], latent)
